# Initial kernel scaffold; baseline (speedup 1.0000x reference)
import functools

import jax
import jax.numpy as jnp
from jax import lax
from jax.experimental import pallas as pl
from jax.experimental.pallas import tpu as pltpu
from jax.experimental.pallas import tpu_sc as plsc

BATCH = 4096
MAX_LEN = 50
EMB = 32
NUM_WORKERS = 32
ITEMS = BATCH // NUM_WORKERS


def _sc_partials(xw, table, w2d):
    mesh = plsc.VectorSubcoreMesh(core_axis_name="c", subcore_axis_name="s")

    @functools.partial(
        pl.kernel,
        out_type=jax.ShapeDtypeStruct((BATCH, EMB), jnp.float32),
        mesh=mesh,
        scratch_types=[
            pltpu.VMEM((MAX_LEN, ITEMS), jnp.int32),
            pltpu.VMEM((ITEMS, EMB), jnp.float32),
            pltpu.VMEM((ITEMS, EMB), jnp.float32),
            pltpu.VMEM((MAX_LEN, EMB), jnp.float32),
        ],
    )
    def sc_kernel(xw_hbm, table_hbm, w_hbm, out_hbm, idx_v, rows_v, acc_v, w_v):
        wid = lax.axis_index("c") * 16 + lax.axis_index("s")
        pltpu.sync_copy(w_hbm, w_v)
        pltpu.sync_copy(xw_hbm.at[wid], idx_v)

        @pl.loop(0, ITEMS)
        def _(j):
            acc_v[j, pl.ds(0, 16)] = jnp.zeros((16,), jnp.float32)
            acc_v[j, pl.ds(16, 16)] = jnp.zeros((16,), jnp.float32)

        @pl.loop(0, MAX_LEN)
        def _(l):
            pltpu.sync_copy(table_hbm.at[idx_v.at[l]], rows_v)
            wl0 = w_v[l, pl.ds(0, 16)]
            wl1 = w_v[l, pl.ds(16, 16)]

            @pl.loop(0, ITEMS)
            def _(j):
                r0 = rows_v[j, pl.ds(0, 16)]
                r1 = rows_v[j, pl.ds(16, 16)]
                plsc.addupdate(acc_v.at[j, pl.ds(0, 16)], r0 * wl0)
                plsc.addupdate(acc_v.at[j, pl.ds(16, 16)], r1 * wl1)

        pltpu.sync_copy(acc_v, out_hbm.at[pl.ds(wid * ITEMS, ITEMS)])

    return sc_kernel(xw, table, w2d)


def _tc_finish_body(p_ref, b_ref, o_ref):
    s = jnp.sum(p_ref[...], axis=1, keepdims=True) + b_ref[0, 0]
    o_ref[...] = jax.nn.sigmoid(s)


def kernel(x, table, W, b):
    xw = x.astype(jnp.int32).reshape(NUM_WORKERS, ITEMS, MAX_LEN)
    xw = jnp.transpose(xw, (0, 2, 1))
    w2d = W.astype(jnp.float32).reshape(MAX_LEN, EMB)

    partials = _sc_partials(xw, table, w2d)

    out = pl.pallas_call(
        _tc_finish_body,
        out_shape=jax.ShapeDtypeStruct((BATCH, 1), jnp.float32),
    )(partials, b.reshape(1, 1))
    return out


# speedup vs baseline: 9.0086x; 9.0086x over previous
import functools

import jax
import jax.numpy as jnp
from jax import lax
from jax.experimental import pallas as pl
from jax.experimental.pallas import tpu as pltpu
from jax.experimental.pallas import tpu_sc as plsc

BATCH = 4096
MAX_LEN = 50
EMB = 32
NUM_WORKERS = 32
ITEMS = BATCH // NUM_WORKERS


def _sc_partials(xw, table, w2d):
    mesh = plsc.VectorSubcoreMesh(core_axis_name="c", subcore_axis_name="s")

    @functools.partial(
        pl.kernel,
        out_type=jax.ShapeDtypeStruct((BATCH, EMB), jnp.float32),
        mesh=mesh,
        scratch_types=[
            pltpu.VMEM((MAX_LEN, ITEMS), jnp.int32),
            pltpu.VMEM((ITEMS, EMB), jnp.float32),
            pltpu.VMEM((ITEMS, EMB), jnp.float32),
            pltpu.VMEM((MAX_LEN, EMB), jnp.float32),
        ],
        compiler_params=pltpu.CompilerParams(use_tc_tiling_on_sc=False),
    )
    def sc_kernel(xw_hbm, table_hbm, w_hbm, out_hbm, idx_v, rows_v, acc_v, w_v):
        wid = lax.axis_index("c") * 16 + lax.axis_index("s")
        pltpu.sync_copy(w_hbm, w_v)
        pltpu.sync_copy(xw_hbm.at[wid], idx_v)

        @pl.loop(0, ITEMS)
        def _(j):
            acc_v[j, pl.ds(0, 16)] = jnp.zeros((16,), jnp.float32)
            acc_v[j, pl.ds(16, 16)] = jnp.zeros((16,), jnp.float32)

        @pl.loop(0, MAX_LEN)
        def _(l):
            pltpu.sync_copy(table_hbm.at[idx_v.at[l]], rows_v)
            wl0 = w_v[l, pl.ds(0, 16)]
            wl1 = w_v[l, pl.ds(16, 16)]

            @pl.loop(0, ITEMS)
            def _(j):
                r0 = rows_v[j, pl.ds(0, 16)]
                r1 = rows_v[j, pl.ds(16, 16)]
                plsc.addupdate(acc_v.at[j, pl.ds(0, 16)], r0 * wl0)
                plsc.addupdate(acc_v.at[j, pl.ds(16, 16)], r1 * wl1)

        pltpu.sync_copy(acc_v, out_hbm.at[pl.ds(wid * ITEMS, ITEMS)])

    return sc_kernel(xw, table, w2d)


def _tc_finish_body(p_ref, b_ref, o_ref):
    s = jnp.sum(p_ref[...], axis=1, keepdims=True) + b_ref[0, 0]
    o_ref[...] = jax.nn.sigmoid(s)


def kernel(x, table, W, b):
    xw = x.astype(jnp.int32).reshape(NUM_WORKERS, ITEMS, MAX_LEN)
    xw = jnp.transpose(xw, (0, 2, 1))
    w2d = W.astype(jnp.float32).reshape(MAX_LEN, EMB)

    partials = _sc_partials(xw, table, w2d)

    out = pl.pallas_call(
        _tc_finish_body,
        out_shape=jax.ShapeDtypeStruct((BATCH, 1), jnp.float32),
    )(partials, b.reshape(1, 1))
    return out


# speedup vs baseline: 9.7774x; 1.0853x over previous
import functools

import jax
import jax.numpy as jnp
from jax import lax
from jax.experimental import pallas as pl
from jax.experimental.pallas import tpu as pltpu
from jax.experimental.pallas import tpu_sc as plsc

BATCH = 4096
MAX_LEN = 50
EMB = 32
NUM_WORKERS = 32
ITEMS = BATCH // NUM_WORKERS


def _sc_partials(xw, table, w2d):
    mesh = plsc.VectorSubcoreMesh(core_axis_name="c", subcore_axis_name="s")

    NBUF = 5

    @functools.partial(
        pl.kernel,
        out_type=jax.ShapeDtypeStruct((BATCH, EMB), jnp.float32),
        mesh=mesh,
        scratch_types=[
            pltpu.VMEM((MAX_LEN, ITEMS), jnp.int32),
            pltpu.VMEM((ITEMS, EMB), jnp.float32),
            pltpu.VMEM((MAX_LEN, EMB), jnp.float32),
        ]
        + [pltpu.VMEM((ITEMS, EMB), jnp.float32) for _ in range(NBUF)]
        + [pltpu.SemaphoreType.DMA for _ in range(NBUF)],
        compiler_params=pltpu.CompilerParams(use_tc_tiling_on_sc=False),
    )
    def sc_kernel(xw_hbm, table_hbm, w_hbm, out_hbm, idx_v, acc_v, w_v, *bufs):
        rows = bufs[:NBUF]
        sems = bufs[NBUF:]
        wid = lax.axis_index("c") * 16 + lax.axis_index("s")
        pltpu.sync_copy(w_hbm, w_v)
        pltpu.sync_copy(xw_hbm.at[wid], idx_v)

        def gdesc(l, b):
            return pltpu.make_async_copy(
                table_hbm.at[idx_v.at[l]], rows[b], sems[b])

        for b in range(NBUF):
            gdesc(b, b).start()

        @pl.loop(0, ITEMS)
        def _(j):
            acc_v[j, pl.ds(0, 16)] = jnp.zeros((16,), jnp.float32)
            acc_v[j, pl.ds(16, 16)] = jnp.zeros((16,), jnp.float32)

        @pl.loop(0, MAX_LEN, step=NBUF)
        def _(base):
            for b in range(NBUF):
                l = base + b
                gdesc(l, b).wait()
                wl0 = w_v[l, pl.ds(0, 16)]
                wl1 = w_v[l, pl.ds(16, 16)]
                rows_b = rows[b]

                @pl.loop(0, ITEMS, step=4)
                def _(j):
                    for u in range(4):
                        r0 = rows_b[j + u, pl.ds(0, 16)]
                        r1 = rows_b[j + u, pl.ds(16, 16)]
                        plsc.addupdate(acc_v.at[j + u, pl.ds(0, 16)], r0 * wl0)
                        plsc.addupdate(acc_v.at[j + u, pl.ds(16, 16)], r1 * wl1)

                @pl.when(l + NBUF < MAX_LEN)
                def _():
                    gdesc(l + NBUF, b).start()

        pltpu.sync_copy(acc_v, out_hbm.at[pl.ds(wid * ITEMS, ITEMS)])

    return sc_kernel(xw, table, w2d)


def _tc_finish_body(p_ref, b_ref, o_ref):
    s = jnp.sum(p_ref[...], axis=1, keepdims=True) + b_ref[0, 0]
    o_ref[...] = jax.nn.sigmoid(s)


def kernel(x, table, W, b):
    xw = x.astype(jnp.int32).reshape(NUM_WORKERS, ITEMS, MAX_LEN)
    xw = jnp.transpose(xw, (0, 2, 1))
    w2d = W.astype(jnp.float32).reshape(MAX_LEN, EMB)

    partials = _sc_partials(xw, table, w2d)

    out = pl.pallas_call(
        _tc_finish_body,
        out_shape=jax.ShapeDtypeStruct((BATCH, 1), jnp.float32),
    )(partials, b.reshape(1, 1))
    return out
